# baseline (speedup 1.0000x reference)
import functools

import jax
import jax.numpy as jnp
from jax import lax
from jax.experimental import pallas as pl
from jax.experimental.pallas import tpu as pltpu
from jax.experimental.pallas import tpu_sc as plsc

_ROWS, _COLS = 4096, 16384
_NC, _NS = 2, 16
_NW = _NC * _NS
_RPW = _ROWS // _NW
_LANES = 16
_BLK = 4 * _LANES
_BLOCKS = _COLS // _BLK


def _mask_row(in_ref, out_ref):
    lanes4 = lax.iota(jnp.int32, _LANES) * 4

    def block(blk, carry):
        ia = blk * _BLK + lanes4
        ib = ia + 1
        ic = ia + 2
        id_ = ia + 3
        a = plsc.load_gather(in_ref, [ia])
        b = plsc.load_gather(in_ref, [ib])
        c = plsc.load_gather(in_ref, [ic])
        d = plsc.load_gather(in_ref, [id_])
        n1 = jnp.where(a >= b, 1, 0)
        n2 = jnp.where(a >= c, 1, 0)
        n3 = jnp.where(a >= d, 1, 0)
        n4 = jnp.where(b >= c, 1, 0)
        n5 = jnp.where(b >= d, 1, 0)
        n6 = jnp.where(c >= d, 1, 0)
        ka = n1 + n2 + n3 >= 2
        kb = n4 + n5 - n1 >= 1
        kc = n6 - n2 - n4 >= 0
        kd = n3 + n5 + n6 <= 1
        one = jnp.float32(1.0)
        zero = jnp.float32(0.0)
        plsc.store_scatter(out_ref, [ia], jnp.where(ka, one, zero))
        plsc.store_scatter(out_ref, [ib], jnp.where(kb, one, zero))
        plsc.store_scatter(out_ref, [ic], jnp.where(kc, one, zero))
        plsc.store_scatter(out_ref, [id_], jnp.where(kd, one, zero))
        return carry

    lax.fori_loop(0, _BLOCKS, block, 0, unroll=2)


@functools.partial(
    pl.kernel,
    out_type=jax.ShapeDtypeStruct((_ROWS, _COLS), jnp.float32),
    mesh=plsc.VectorSubcoreMesh(core_axis_name="c", subcore_axis_name="s"),
    compiler_params=pltpu.CompilerParams(needs_layout_passes=False),
    scratch_types=[
        pltpu.VMEM((_COLS,), jnp.float32),
        pltpu.VMEM((_COLS,), jnp.float32),
        pltpu.VMEM((_COLS,), jnp.float32),
        pltpu.VMEM((_COLS,), jnp.float32),
        pltpu.SemaphoreType.DMA,
        pltpu.SemaphoreType.DMA,
        pltpu.SemaphoreType.DMA,
        pltpu.SemaphoreType.DMA,
    ],
)
def _ampere_mask(in_hbm, out_hbm, ib0, ib1, ob0, ob1, is0, is1, os0, os1):
    wid = lax.axis_index("s") * _NC + lax.axis_index("c")
    row0 = wid * _RPW
    ibufs, obufs = (ib0, ib1), (ob0, ob1)
    isems, osems = (is0, is1), (os0, os1)

    pltpu.async_copy(in_hbm.at[row0], ibufs[0], isems[0])
    pltpu.async_copy(in_hbm.at[row0 + 1], ibufs[1], isems[1])

    def pair_step(j, carry):
        for s in range(2):
            i = 2 * j + s
            r = row0 + i
            pltpu.make_async_copy(in_hbm.at[r], ibufs[s], isems[s]).wait()

            @pl.when(j > 0)
            def _wait_prev_out():
                pltpu.make_async_copy(
                    obufs[s], out_hbm.at[r - 2], osems[s]
                ).wait()

            _mask_row(ibufs[s], obufs[s])
            pltpu.async_copy(obufs[s], out_hbm.at[r], osems[s])

            @pl.when(i + 2 < _RPW)
            def _prefetch_next_in():
                pltpu.async_copy(in_hbm.at[r + 2], ibufs[s], isems[s])

        return carry

    lax.fori_loop(0, _RPW // 2, pair_step, 0)
    for s in range(2):
        r = row0 + _RPW - 2 + s
        pltpu.make_async_copy(obufs[s], out_hbm.at[r], osems[s]).wait()


def kernel(mask_scores, ampere_temperature):
    del ampere_temperature
    return _ampere_mask(mask_scores)


# speedup vs baseline: 1.0256x; 1.0256x over previous
import functools

import jax
import jax.numpy as jnp
from jax import lax
from jax.experimental import pallas as pl
from jax.experimental.pallas import tpu as pltpu
from jax.experimental.pallas import tpu_sc as plsc

_ROWS, _COLS = 4096, 16384
_NC, _NS = 2, 16
_NW = _NC * _NS
_RPW = _ROWS // _NW
_LANES = 16
_BLK = 4 * _LANES
_BLOCKS = _COLS // _BLK


def _mask_row(in_ref, out_ref):
    ia = lax.iota(jnp.int32, _LANES) * 4
    ib = ia + 1
    ic = ia + 2
    id_ = ia + 3
    one = jnp.float32(1.0)
    zero = jnp.float32(0.0)

    def block(blk, carry):
        base = blk * _BLK
        in_blk = in_ref.at[pl.ds(base, _BLK)]
        out_blk = out_ref.at[pl.ds(base, _BLK)]
        a = plsc.load_gather(in_blk, [ia])
        b = plsc.load_gather(in_blk, [ib])
        c = plsc.load_gather(in_blk, [ic])
        d = plsc.load_gather(in_blk, [id_])
        x1 = a >= b
        x6 = c >= d
        h1 = jnp.where(x1, a, b)
        l1 = jnp.where(x1, b, a)
        h2 = jnp.where(x6, c, d)
        l2 = jnp.where(x6, d, c)
        y = h1 >= h2
        u = l1 >= h2
        v = h1 >= l2
        kh1 = y | v
        kl1 = y & u
        h1o = jnp.where(kh1, one, zero)
        l1o = jnp.where(kl1, one, zero)
        h2o = jnp.where(kl1, zero, one)
        l2o = jnp.where(kh1, zero, one)
        plsc.store_scatter(out_blk, [ia], jnp.where(x1, h1o, l1o))
        plsc.store_scatter(out_blk, [ib], jnp.where(x1, l1o, h1o))
        plsc.store_scatter(out_blk, [ic], jnp.where(x6, h2o, l2o))
        plsc.store_scatter(out_blk, [id_], jnp.where(x6, l2o, h2o))
        return carry

    lax.fori_loop(0, _BLOCKS, block, 0)


@functools.partial(
    pl.kernel,
    out_type=jax.ShapeDtypeStruct((_ROWS, _COLS), jnp.float32),
    mesh=plsc.VectorSubcoreMesh(core_axis_name="c", subcore_axis_name="s"),
    compiler_params=pltpu.CompilerParams(needs_layout_passes=False),
    scratch_types=[
        pltpu.VMEM((_COLS,), jnp.float32),
        pltpu.VMEM((_COLS,), jnp.float32),
        pltpu.VMEM((_COLS,), jnp.float32),
        pltpu.VMEM((_COLS,), jnp.float32),
        pltpu.SemaphoreType.DMA,
        pltpu.SemaphoreType.DMA,
        pltpu.SemaphoreType.DMA,
        pltpu.SemaphoreType.DMA,
    ],
)
def _ampere_mask(in_hbm, out_hbm, ib0, ib1, ob0, ob1, is0, is1, os0, os1):
    wid = lax.axis_index("s") * _NC + lax.axis_index("c")
    row0 = wid * _RPW
    ibufs, obufs = (ib0, ib1), (ob0, ob1)
    isems, osems = (is0, is1), (os0, os1)

    pltpu.async_copy(in_hbm.at[row0], ibufs[0], isems[0])
    pltpu.async_copy(in_hbm.at[row0 + 1], ibufs[1], isems[1])

    def pair_step(j, carry):
        for s in range(2):
            i = 2 * j + s
            r = row0 + i
            pltpu.make_async_copy(in_hbm.at[r], ibufs[s], isems[s]).wait()

            @pl.when(j > 0)
            def _wait_prev_out():
                pltpu.make_async_copy(
                    obufs[s], out_hbm.at[r - 2], osems[s]
                ).wait()

            _mask_row(ibufs[s], obufs[s])
            pltpu.async_copy(obufs[s], out_hbm.at[r], osems[s])

            @pl.when(i + 2 < _RPW)
            def _prefetch_next_in():
                pltpu.async_copy(in_hbm.at[r + 2], ibufs[s], isems[s])

        return carry

    lax.fori_loop(0, _RPW // 2, pair_step, 0)
    for s in range(2):
        r = row0 + _RPW - 2 + s
        pltpu.make_async_copy(obufs[s], out_hbm.at[r], osems[s]).wait()


def kernel(mask_scores, ampere_temperature):
    del ampere_temperature
    return _ampere_mask(mask_scores)
